# baseline (speedup 1.0000x reference)
import functools

import jax
import jax.numpy as jnp
from jax import lax
from jax.experimental import pallas as pl
from jax.experimental.pallas import tpu as pltpu
from jax.experimental.pallas import tpu_sc as plsc

_N = 10000
_E = 320000
_D = 128
_NC = 2
_NS = 16
_NW = _NC * _NS
_EPT = _E // _NW
_C = 112
_EPTP = 10080
_NCH = _EPTP // _C
_EPAD = _NW * _EPTP - _E
_RPT = 640
_NPAD = 10240


def _make_hist():
    mesh = plsc.VectorSubcoreMesh(core_axis_name="c", subcore_axis_name="s")

    @functools.partial(
        pl.kernel,
        out_type=jax.ShapeDtypeStruct((_NW, _NPAD), jnp.float32),
        mesh=mesh,
        scratch_types=[
            pltpu.VMEM((_NCH, _C), jnp.int32),
            pltpu.VMEM((_NPAD,), jnp.float32),
        ],
        compiler_params=pltpu.CompilerParams(needs_layout_passes=False),
    )
    def hist_kernel(dst_hbm, out_hbm, dstv, hist):
        cid = lax.axis_index("c")
        sid = lax.axis_index("s")
        wid = cid * _NS + sid
        pltpu.sync_copy(dst_hbm.at[wid], dstv)
        zeros16 = jnp.zeros((16,), jnp.float32)

        def zstep(i, carry):
            hist[pl.ds(i * 16, 16)] = zeros16
            return carry

        lax.fori_loop(0, _NPAD // 16, zstep, 0)
        ones16 = jnp.ones((16,), jnp.float32)
        npv = _C // 16

        def astep(i, carry):
            row = i // npv
            j = i % npv
            idx = dstv[row, pl.ds(j * 16, 16)]
            plsc.addupdate_scatter(hist, [idx], ones16)
            return carry

        lax.fori_loop(0, _NCH * npv, astep, 0)
        pltpu.sync_copy(hist, out_hbm.at[wid])

    return hist_kernel


def _make_agg():
    mesh = plsc.VectorSubcoreMesh(core_axis_name="c", subcore_axis_name="s")

    @functools.partial(
        pl.kernel,
        out_type=jax.ShapeDtypeStruct((_NC, _NPAD, _D), jnp.float32),
        mesh=mesh,
        scratch_types=[
            pltpu.VMEM((_NCH, _C), jnp.int32),
            pltpu.VMEM((_NCH, _C), jnp.int32),
            pltpu.VMEM((_C, _D), jnp.float32),
            pltpu.VMEM_SHARED((_NPAD, _D), jnp.float32),
            pltpu.SemaphoreType.DMA,
        ],
        compiler_params=pltpu.CompilerParams(needs_layout_passes=False),
    )
    def agg_kernel(h_hbm, src_hbm, dst_hbm, zero_hbm, out_hbm,
                   srcv, dstv, rows0, acc, gsem):
        cid = lax.axis_index("c")
        sid = lax.axis_index("s")
        wid = cid * _NS + sid
        pltpu.sync_copy(src_hbm.at[wid], srcv)
        pltpu.sync_copy(dst_hbm.at[wid], dstv)
        pltpu.sync_copy(zero_hbm, acc.at[pl.ds(sid * _RPT, _RPT)])
        plsc.subcore_barrier()

        def step(i, carry):
            pltpu.async_copy(h_hbm.at[srcv.at[i]], rows0, gsem).wait()
            pltpu.sync_copy(rows0, acc.at[dstv.at[i]], add=True)
            return carry

        lax.fori_loop(0, _NCH, step, 0)
        plsc.subcore_barrier()
        pltpu.sync_copy(acc.at[pl.ds(sid * _RPT, _RPT)],
                        out_hbm.at[cid, pl.ds(sid * _RPT, _RPT)])

    return agg_kernel


def _tc_layer_body(relu, head, h_ref, p_ref, histT_ref, wl_ref, bl_ref,
                   wr_ref, g_ref, be_ref, *rest):
    if head:
        wh1_ref, bh1_ref, wh2_ref, bh2_ref, o_ref = rest
    else:
        (o_ref,) = rest
    deg = jnp.sum(histT_ref[...], axis=1, keepdims=True)[:_N]
    deg = jnp.maximum(deg, 1.0)
    agg = (p_ref[0] + p_ref[1])[:_N] / deg
    h = h_ref[...]
    z = (lax.dot_general(agg, wl_ref[...], (((1,), (1,)), ((), ())),
                         preferred_element_type=jnp.float32)
         + lax.dot_general(h, wr_ref[...], (((1,), (1,)), ((), ())),
                           preferred_element_type=jnp.float32)
         + bl_ref[...])
    mu = jnp.mean(z, axis=0, keepdims=True)
    zc = z - mu
    var = jnp.mean(zc * zc, axis=0, keepdims=True)
    hn = zc * lax.rsqrt(var + 1e-5) * g_ref[...] + be_ref[...]
    if relu:
        hn = jnp.maximum(hn, 0.0)
    hn = hn + h
    if not head:
        o_ref[...] = hn
        return
    a1 = lax.dot_general(hn, wh1_ref[...], (((1,), (1,)), ((), ())),
                         preferred_element_type=jnp.float32) + bh1_ref[...]
    a1 = jnp.maximum(a1, 0.0)
    a2 = jnp.sum(a1 * wh2_ref[...], axis=1, keepdims=True) + bh2_ref[...]
    o_ref[...] = jax.nn.sigmoid(a2)


def _tc_layer(h, parts, histT, wl, bl, wr, g, be, relu):
    return pl.pallas_call(
        functools.partial(_tc_layer_body, relu, False),
        out_shape=jax.ShapeDtypeStruct((_N, _D), jnp.float32),
    )(h, parts, histT, wl, bl, wr, g, be)


def _tc_layer_head(h, parts, histT, wl, bl, wr, g, be, wh1, bh1, wh2, bh2):
    return pl.pallas_call(
        functools.partial(_tc_layer_body, False, True),
        out_shape=jax.ShapeDtypeStruct((_N, 1), jnp.float32),
    )(h, parts, histT, wl, bl, wr, g, be, wh1, bh1, wh2, bh2)


_hist_fn = _make_hist()
_agg_fn = _make_agg()


def kernel(x, edge_index, Wl0, bl0, Wr0, g0, be0, Wl1, bl1, Wr1, g1, be1,
           Wl2, bl2, Wr2, g2, be2, Wh1, bh1, Wh2, bh2):
    pad_src = jnp.zeros((_EPAD,), jnp.int32)
    pad_dst = _N + (jnp.arange(_EPAD, dtype=jnp.int32) % (_NPAD - _N))
    src = jnp.concatenate([edge_index[0], pad_src]).reshape(_NW, _NCH, _C)
    dst = jnp.concatenate([edge_index[1], pad_dst]).reshape(_NW, _NCH, _C)
    zero = jnp.zeros((_RPT, _D), jnp.float32)
    histT = _hist_fn(dst).T

    def r2(v):
        return v.reshape(1, -1)

    h = x
    for i, (Wl, bl, Wr, g, be) in enumerate(
            [(Wl0, bl0, Wr0, g0, be0), (Wl1, bl1, Wr1, g1, be1),
             (Wl2, bl2, Wr2, g2, be2)]):
        parts = _agg_fn(h, src, dst, zero)
        if i != 2:
            h = _tc_layer(h, parts, histT, Wl, r2(bl), Wr, r2(g), r2(be),
                          relu=True)
        else:
            out = _tc_layer_head(h, parts, histT, Wl, r2(bl), Wr, r2(g),
                                 r2(be), Wh1, r2(bh1), Wh2, r2(bh2))
    return out


# speedup vs baseline: 1.4065x; 1.4065x over previous
import functools

import jax
import jax.numpy as jnp
from jax import lax
from jax.experimental import pallas as pl
from jax.experimental.pallas import tpu as pltpu
from jax.experimental.pallas import tpu_sc as plsc

_N = 10000
_E = 320000
_D = 128
_NC = 2
_NS = 16
_NW = _NC * _NS
_EPT = _E // _NW
_C = 80
_NCH = _EPT // _C
_RPT = 640
_NPAD = 10240


def _make_hist():
    mesh = plsc.VectorSubcoreMesh(core_axis_name="c", subcore_axis_name="s")

    @functools.partial(
        pl.kernel,
        out_type=jax.ShapeDtypeStruct((_NW, _NPAD), jnp.float32),
        mesh=mesh,
        scratch_types=[
            pltpu.VMEM((_NCH, _C), jnp.int32),
            pltpu.VMEM((_NPAD,), jnp.float32),
        ],
        compiler_params=pltpu.CompilerParams(needs_layout_passes=False),
    )
    def hist_kernel(dst_hbm, out_hbm, dstv, hist):
        cid = lax.axis_index("c")
        sid = lax.axis_index("s")
        wid = cid * _NS + sid
        pltpu.sync_copy(dst_hbm.at[wid], dstv)
        zeros16 = jnp.zeros((16,), jnp.float32)

        def zstep(i, carry):
            hist[pl.ds(i * 16, 16)] = zeros16
            return carry

        lax.fori_loop(0, _NPAD // 16, zstep, 0)
        ones16 = jnp.ones((16,), jnp.float32)
        npv = _C // 16

        def astep(i, carry):
            row = i // npv
            j = i % npv
            idx = dstv[row, pl.ds(j * 16, 16)]
            plsc.addupdate_scatter(hist, [idx], ones16)
            return carry

        lax.fori_loop(0, _NCH * npv, astep, 0)
        pltpu.sync_copy(hist, out_hbm.at[wid])

    return hist_kernel


def _make_agg():
    mesh = plsc.VectorSubcoreMesh(core_axis_name="c", subcore_axis_name="s")

    @functools.partial(
        pl.kernel,
        out_type=jax.ShapeDtypeStruct((_NC, _NPAD, _D), jnp.float32),
        mesh=mesh,
        scratch_types=[
            pltpu.VMEM((_NCH, _C), jnp.int32),
            pltpu.VMEM((_NCH, _C), jnp.int32),
            pltpu.VMEM((_C, _D), jnp.float32),
            pltpu.VMEM_SHARED((_NPAD, _D), jnp.float32),
            pltpu.SemaphoreType.DMA,
        ],
        compiler_params=pltpu.CompilerParams(needs_layout_passes=False),
    )
    def agg_kernel(h_hbm, src_hbm, dst_hbm, zero_hbm, out_hbm,
                   srcv, dstv, rows0, acc, gsem):
        cid = lax.axis_index("c")
        sid = lax.axis_index("s")
        wid = cid * _NS + sid
        pltpu.sync_copy(src_hbm.at[wid], srcv)
        pltpu.sync_copy(dst_hbm.at[wid], dstv)
        pltpu.sync_copy(zero_hbm, acc.at[pl.ds(sid * _RPT, _RPT)])
        plsc.subcore_barrier()

        def step(i, carry):
            pltpu.async_copy(h_hbm.at[srcv.at[i]], rows0, gsem).wait()
            pltpu.sync_copy(rows0, acc.at[dstv.at[i]], add=True)
            return carry

        lax.fori_loop(0, _NCH, step, 0)
        plsc.subcore_barrier()
        pltpu.sync_copy(acc.at[pl.ds(sid * _RPT, _RPT)],
                        out_hbm.at[cid, pl.ds(sid * _RPT, _RPT)])

    return agg_kernel


def _tc_layer_body(relu, head, h_ref, p_ref, histT_ref, wl_ref, bl_ref,
                   wr_ref, g_ref, be_ref, *rest):
    if head:
        wh1_ref, bh1_ref, wh2_ref, bh2_ref, o_ref = rest
    else:
        (o_ref,) = rest
    deg = jnp.sum(histT_ref[...], axis=1, keepdims=True)[:_N]
    deg = jnp.maximum(deg, 1.0)
    agg = (p_ref[0] + p_ref[1])[:_N] / deg
    h = h_ref[...]
    z = (lax.dot_general(agg, wl_ref[...], (((1,), (1,)), ((), ())),
                         preferred_element_type=jnp.float32)
         + lax.dot_general(h, wr_ref[...], (((1,), (1,)), ((), ())),
                           preferred_element_type=jnp.float32)
         + bl_ref[...])
    mu = jnp.mean(z, axis=0, keepdims=True)
    zc = z - mu
    var = jnp.mean(zc * zc, axis=0, keepdims=True)
    hn = zc * lax.rsqrt(var + 1e-5) * g_ref[...] + be_ref[...]
    if relu:
        hn = jnp.maximum(hn, 0.0)
    hn = hn + h
    if not head:
        o_ref[...] = hn
        return
    a1 = lax.dot_general(hn, wh1_ref[...], (((1,), (1,)), ((), ())),
                         preferred_element_type=jnp.float32) + bh1_ref[...]
    a1 = jnp.maximum(a1, 0.0)
    a2 = jnp.sum(a1 * wh2_ref[...], axis=1, keepdims=True) + bh2_ref[...]
    o_ref[...] = jax.nn.sigmoid(a2)


def _tc_layer(h, parts, histT, wl, bl, wr, g, be, relu):
    return pl.pallas_call(
        functools.partial(_tc_layer_body, relu, False),
        out_shape=jax.ShapeDtypeStruct((_N, _D), jnp.float32),
    )(h, parts, histT, wl, bl, wr, g, be)


def _tc_layer_head(h, parts, histT, wl, bl, wr, g, be, wh1, bh1, wh2, bh2):
    return pl.pallas_call(
        functools.partial(_tc_layer_body, False, True),
        out_shape=jax.ShapeDtypeStruct((_N, 1), jnp.float32),
    )(h, parts, histT, wl, bl, wr, g, be, wh1, bh1, wh2, bh2)


_hist_fn = _make_hist()
_agg_fn = _make_agg()


def kernel(x, edge_index, Wl0, bl0, Wr0, g0, be0, Wl1, bl1, Wr1, g1, be1,
           Wl2, bl2, Wr2, g2, be2, Wh1, bh1, Wh2, bh2):
    src = edge_index[0].reshape(_NW, _NCH, _C)
    dst = edge_index[1].reshape(_NW, _NCH, _C)
    zero = jnp.zeros((_RPT, _D), jnp.float32)
    histT = _hist_fn(dst).T

    def r2(v):
        return v.reshape(1, -1)

    h = x
    for i, (Wl, bl, Wr, g, be) in enumerate(
            [(Wl0, bl0, Wr0, g0, be0), (Wl1, bl1, Wr1, g1, be1),
             (Wl2, bl2, Wr2, g2, be2)]):
        parts = _agg_fn(h, src, dst, zero)
        if i != 2:
            h = _tc_layer(h, parts, histT, Wl, r2(bl), Wr, r2(g), r2(be),
                          relu=True)
        else:
            out = _tc_layer_head(h, parts, histT, Wl, r2(bl), Wr, r2(g),
                                 r2(be), Wh1, r2(bh1), Wh2, r2(bh2))
    return out
